# baseline (speedup 1.0000x reference)
import functools

import jax
import jax.numpy as jnp
from jax import lax
from jax.experimental import pallas as pl
from jax.experimental.pallas import tpu as pltpu
from jax.experimental.pallas import tpu_sc as plsc

VOCAB = 1000000
EMBED_DIM = 16
BATCH = 16384
N_FIELDS = 26

_NW = 32
_B_PER_W = BATCH // _NW
_CB = 64
_NCHUNK = _B_PER_W // _CB
_CIDX = _CB * N_FIELDS


def _sig_kernel(table_hbm, xt_hbm, out_hbm, idx_v, rows_v, out_v, sem):
    wid = lax.axis_index("s") * 2 + lax.axis_index("c")
    base = wid * _B_PER_W
    lanes = jnp.arange(16, dtype=jnp.int32)
    zeros = jnp.zeros((16,), jnp.int32)
    for c in range(_NCHUNK):
        b0 = base + c * _CB
        for f in range(N_FIELDS):
            pltpu.sync_copy(xt_hbm.at[f, pl.ds(b0, _CB)],
                            idx_v.at[pl.ds(f * _CB, _CB)])
        pltpu.async_copy(table_hbm.at[idx_v], rows_v, sem).wait()

        def body(bb, carry):
            vb = zeros + bb
            for f in range(N_FIELDS):
                r = rows_v[f * _CB + bb]
                s = 1.0 / (1.0 + jnp.exp(-r))
                plsc.store_scatter(
                    out_v, [jnp.full((16,), f, jnp.int32), lanes, vb], s)
            return carry

        lax.fori_loop(0, _CB, body, 0)
        pltpu.sync_copy(out_v, out_hbm.at[:, :, pl.ds(b0, _CB)])


@jax.jit
def _run(table, xt):
    mesh = plsc.VectorSubcoreMesh(core_axis_name="c", subcore_axis_name="s")
    f = functools.partial(
        pl.kernel,
        mesh=mesh,
        out_type=jax.ShapeDtypeStruct((N_FIELDS, EMBED_DIM, BATCH), jnp.float32),
        scratch_types=[
            pltpu.VMEM((_CIDX,), jnp.int32),
            pltpu.VMEM((_CIDX, EMBED_DIM), jnp.float32),
            pltpu.VMEM((N_FIELDS, EMBED_DIM, _CB), jnp.float32),
            pltpu.SemaphoreType.DMA,
        ],
        compiler_params=pltpu.CompilerParams(
            use_tc_tiling_on_sc=False, needs_layout_passes=False),
    )(_sig_kernel)
    return f(table, xt)


def kernel(table, x):
    out_t = _run(table, x.T)
    return jnp.transpose(out_t, (2, 0, 1))


# speedup vs baseline: 1.1005x; 1.1005x over previous
import functools

import jax
import jax.numpy as jnp
from jax import lax
from jax.experimental import pallas as pl
from jax.experimental.pallas import tpu as pltpu
from jax.experimental.pallas import tpu_sc as plsc

VOCAB = 1000000
EMBED_DIM = 16
BATCH = 16384
N_FIELDS = 26

_NW = 32
_B_PER_W = BATCH // _NW
_CB = 64
_NCHUNK = _B_PER_W // _CB
_CIDX = _CB * N_FIELDS


def _sig_kernel(table_hbm, xt_hbm, out_hbm, idx2_v, idx_v, rows_v, out_v, sem):
    wid = lax.axis_index("s") * 2 + lax.axis_index("c")
    base = wid * _B_PER_W
    lanes = jnp.arange(16, dtype=jnp.int32)
    zeros = jnp.zeros((16,), jnp.int32)
    for c in range(_NCHUNK):
        b0 = base + c * _CB
        pltpu.sync_copy(xt_hbm.at[:, pl.ds(b0, _CB)], idx2_v)
        for f in range(N_FIELDS):
            for k in range(_CB // 16):
                v = plsc.bitcast(idx2_v[f, pl.ds(k * 16, 16)], jnp.int32)
                idx_v[pl.ds(f * _CB + k * 16, 16)] = v
        pltpu.async_copy(table_hbm.at[idx_v], rows_v, sem).wait()

        def body(bb, carry):
            vb = zeros + bb
            for f in range(N_FIELDS):
                r = rows_v[f * _CB + bb]
                s = 1.0 / (1.0 + jnp.exp(-r))
                plsc.store_scatter(
                    out_v, [jnp.full((16,), f, jnp.int32), lanes, vb], s)
            return carry

        lax.fori_loop(0, _CB, body, 0)
        pltpu.sync_copy(out_v, out_hbm.at[:, :, pl.ds(b0, _CB)])


@jax.jit
def _run(table, xt):
    mesh = plsc.VectorSubcoreMesh(core_axis_name="c", subcore_axis_name="s")
    f = functools.partial(
        pl.kernel,
        mesh=mesh,
        out_type=jax.ShapeDtypeStruct((N_FIELDS, EMBED_DIM, BATCH), jnp.float32),
        scratch_types=[
            pltpu.VMEM((N_FIELDS, _CB), jnp.float32),
            pltpu.VMEM((_CIDX,), jnp.int32),
            pltpu.VMEM((_CIDX, EMBED_DIM), jnp.float32),
            pltpu.VMEM((N_FIELDS, EMBED_DIM, _CB), jnp.float32),
            pltpu.SemaphoreType.DMA,
        ],
        compiler_params=pltpu.CompilerParams(
            use_tc_tiling_on_sc=False, needs_layout_passes=False),
    )(_sig_kernel)
    return f(table, xt)


def kernel(table, x):
    xf = lax.bitcast_convert_type(x, jnp.float32)
    out_t = _run(table, xf.T)
    return jnp.transpose(out_t, (2, 0, 1))


# speedup vs baseline: 1.2092x; 1.0987x over previous
import functools

import jax
import jax.numpy as jnp
from jax import lax
from jax.experimental import pallas as pl
from jax.experimental.pallas import tpu as pltpu
from jax.experimental.pallas import tpu_sc as plsc

VOCAB = 1000000
EMBED_DIM = 16
BATCH = 16384
N_FIELDS = 26

_NW = 32
_B_PER_W = BATCH // _NW
_CB = 64
_NCHUNK = _B_PER_W // _CB
_CIDX = _CB * N_FIELDS
_OPAD = 65


def _sig_kernel(table_hbm, xq_hbm, out_hbm, idx4_v, idx_v, rows_v, out_v, sem):
    wid = lax.axis_index("s") * 2 + lax.axis_index("c")
    lanes = jnp.arange(16, dtype=jnp.int32)
    zeros = jnp.zeros((16,), jnp.int32)
    for c in range(_NCHUNK):
        b0 = wid * _B_PER_W + c * _CB
        jt = wid * 4 + c // 2
        c0 = (c % 2) * _CB
        pltpu.sync_copy(xq_hbm.at[:, jt, :, pl.ds(c0, _CB)], idx4_v)
        for f in range(N_FIELDS):
            for k in range(_CB // 16):
                v = idx4_v[f // 8, f % 8, pl.ds(k * 16, 16)]
                idx_v[pl.ds(f * _CB + k * 16, 16)] = v
        pltpu.async_copy(table_hbm.at[idx_v], rows_v, sem).wait()

        def body(bb, carry):
            vb = zeros + bb
            for f in range(N_FIELDS):
                r = rows_v[f * _CB + bb]
                s = 1.0 / (1.0 + jnp.exp(-r))
                plsc.store_scatter(
                    out_v, [jnp.full((16,), f, jnp.int32), lanes, vb], s)
            return carry

        lax.fori_loop(0, _CB, body, 0)
        pltpu.sync_copy(out_v.at[:, :, pl.ds(0, _CB)],
                        out_hbm.at[:, :, pl.ds(b0, _CB)])


@jax.jit
def _run(table, xq):
    mesh = plsc.VectorSubcoreMesh(core_axis_name="c", subcore_axis_name="s")
    f = functools.partial(
        pl.kernel,
        mesh=mesh,
        out_type=jax.ShapeDtypeStruct((N_FIELDS, EMBED_DIM, BATCH), jnp.float32),
        scratch_types=[
            pltpu.VMEM((4, 8, _CB), jnp.int32),
            pltpu.VMEM((_CIDX,), jnp.int32),
            pltpu.VMEM((_CIDX, EMBED_DIM), jnp.float32),
            pltpu.VMEM((N_FIELDS, EMBED_DIM, _OPAD), jnp.float32),
            pltpu.SemaphoreType.DMA,
        ],
        compiler_params=pltpu.CompilerParams(
            use_tc_tiling_on_sc=False, needs_layout_passes=False),
    )(_sig_kernel)
    return f(table, xq)


def kernel(table, x):
    xp = jnp.pad(x, ((0, 0), (0, 32 - N_FIELDS)))
    xq = xp.T.reshape(4, 8, 128, 128).transpose(0, 2, 1, 3)
    out_t = _run(table, xq)
    return jnp.transpose(out_t, (2, 0, 1))
